# baseline (speedup 1.0000x reference)
import functools

import jax
import jax.numpy as jnp
from jax import lax
from jax.experimental import pallas as pl
from jax.experimental.pallas import tpu as pltpu
from jax.experimental.pallas import tpu_sc as plsc

_NC = 2
_NS = 16
_NW = _NC * _NS
_LANES = 16
_BATCH = 16384
_DIM = 128
_BPW = _BATCH // _NW
_CHUNK = 128
_NCHUNK = _BPW // _CHUNK
_VECS = _DIM // _LANES


def _sc_body(ind_hbm, eeg_hbm, ir_hbm, b_hbm, out_hbm,
             idx_v, rows_v, eeg_v, ir_v, acc_v, gsem, esem, isem):
    c = lax.axis_index("c")
    s = lax.axis_index("s")
    wid = c * _NS + s
    base = wid * _BPW

    pltpu.sync_copy(ind_hbm.at[wid], idx_v)

    zero = jnp.zeros((_LANES,), jnp.float32)
    acc_e = zero
    acc_i = zero

    for ch in range(_NCHUNK):
        row0 = base + ch * _CHUNK
        gcopy = pltpu.async_copy(b_hbm.at[idx_v.at[ch]], rows_v, gsem)
        ecopy = pltpu.async_copy(eeg_hbm.at[pl.ds(row0, _CHUNK)], eeg_v, esem)
        icopy = pltpu.async_copy(ir_hbm.at[pl.ds(row0, _CHUNK)], ir_v, isem)
        gcopy.wait()
        ecopy.wait()
        icopy.wait()

        @plsc.parallel_loop(0, _CHUNK, unroll=4, carry=(acc_e, acc_i))
        def _row(r, carry):
            a_e, a_i = carry
            for j in range(_VECS):
                col = j * _LANES
                b = rows_v[r, pl.ds(col, _LANES)]
                e = eeg_v[r, pl.ds(col, _LANES)]
                i = ir_v[r, pl.ds(col, _LANES)]
                de = b - e
                di = b - i
                a_e = a_e + de * de
                a_i = a_i + di * di
            return a_e, a_i

        acc_e, acc_i = _row

    acc_v[...] = acc_e + acc_i
    pltpu.sync_copy(acc_v, out_hbm.at[wid])


@jax.jit
def _quant_loss(ind3, eeg, ir, b):
    mesh = plsc.VectorSubcoreMesh(
        core_axis_name="c", subcore_axis_name="s",
        num_cores=_NC, num_subcores=_NS)
    partials = pl.kernel(
        _sc_body,
        out_type=jax.ShapeDtypeStruct((_NW, _LANES), jnp.float32),
        mesh=mesh,
        scratch_types=[
            pltpu.VMEM((_NCHUNK, _CHUNK), jnp.int32),
            pltpu.VMEM((_CHUNK, _DIM), jnp.float32),
            pltpu.VMEM((_CHUNK, _DIM), jnp.float32),
            pltpu.VMEM((_CHUNK, _DIM), jnp.float32),
            pltpu.VMEM((_LANES,), jnp.float32),
            pltpu.SemaphoreType.DMA,
            pltpu.SemaphoreType.DMA,
            pltpu.SemaphoreType.DMA,
        ],
    )(ind3, eeg, ir, b)
    return jnp.sum(partials)


def kernel(eeg, ir, ind, B, un_eeg, un_ir, device):
    ind3 = ind.astype(jnp.int32).reshape(_NW, _NCHUNK, _CHUNK)
    return _quant_loss(ind3, eeg, ir, B)


# speedup vs baseline: 1.1319x; 1.1319x over previous
import functools

import jax
import jax.numpy as jnp
from jax import lax
from jax.experimental import pallas as pl
from jax.experimental.pallas import tpu as pltpu
from jax.experimental.pallas import tpu_sc as plsc

_NC = 2
_NS = 16
_NW = _NC * _NS
_LANES = 16
_BATCH = 16384
_DIM = 128
_BPW = _BATCH // _NW
_CHUNK = 128
_NCHUNK = _BPW // _CHUNK
_VECS = _DIM // _LANES


def _sc_body(ind_hbm, eeg_hbm, ir_hbm, b_hbm, out_hbm,
             idx_v, rows_v, eeg_v, ir_v, acc_v, sems):
    c = lax.axis_index("c")
    s = lax.axis_index("s")
    wid = c * _NS + s
    base = wid * _BPW

    pltpu.sync_copy(ind_hbm.at[wid], idx_v)

    def fire(ch):
        buf = ch % 2
        row0 = base + ch * _CHUNK
        return (
            pltpu.async_copy(b_hbm.at[idx_v.at[ch]], rows_v.at[buf],
                             sems.at[buf, 0]),
            pltpu.async_copy(eeg_hbm.at[pl.ds(row0, _CHUNK)], eeg_v.at[buf],
                             sems.at[buf, 1]),
            pltpu.async_copy(ir_hbm.at[pl.ds(row0, _CHUNK)], ir_v.at[buf],
                             sems.at[buf, 2]),
        )

    zero = jnp.zeros((_LANES,), jnp.float32)
    acc_e = zero
    acc_i = zero

    inflight = fire(0)
    for ch in range(_NCHUNK):
        buf = ch % 2
        for cp in inflight:
            cp.wait()
        if ch + 1 < _NCHUNK:
            inflight = fire(ch + 1)

        @plsc.parallel_loop(0, _CHUNK, unroll=4, carry=(acc_e, acc_i))
        def _row(r, carry):
            a_e, a_i = carry
            for j in range(_VECS):
                col = j * _LANES
                b = rows_v[buf, r, pl.ds(col, _LANES)]
                e = eeg_v[buf, r, pl.ds(col, _LANES)]
                i = ir_v[buf, r, pl.ds(col, _LANES)]
                de = b - e
                di = b - i
                a_e = a_e + de * de
                a_i = a_i + di * di
            return a_e, a_i

        acc_e, acc_i = _row

    acc_v[...] = acc_e + acc_i
    pltpu.sync_copy(acc_v, out_hbm.at[wid])


@jax.jit
def _quant_loss(ind3, eeg, ir, b):
    mesh = plsc.VectorSubcoreMesh(
        core_axis_name="c", subcore_axis_name="s",
        num_cores=_NC, num_subcores=_NS)
    partials = pl.kernel(
        _sc_body,
        out_type=jax.ShapeDtypeStruct((_NW, _LANES), jnp.float32),
        mesh=mesh,
        scratch_types=[
            pltpu.VMEM((_NCHUNK, _CHUNK), jnp.int32),
            pltpu.VMEM((2, _CHUNK, _DIM), jnp.float32),
            pltpu.VMEM((2, _CHUNK, _DIM), jnp.float32),
            pltpu.VMEM((2, _CHUNK, _DIM), jnp.float32),
            pltpu.VMEM((_LANES,), jnp.float32),
            pltpu.SemaphoreType.DMA((2, 3)),
        ],
    )(ind3, eeg, ir, b)
    return jnp.sum(partials)


def kernel(eeg, ir, ind, B, un_eeg, un_ir, device):
    ind3 = ind.astype(jnp.int32).reshape(_NW, _NCHUNK, _CHUNK)
    return _quant_loss(ind3, eeg, ir, B)
